# baseline (speedup 1.0000x reference)
import jax
import jax.numpy as jnp
from jax import lax
from jax.experimental import pallas as pl
from jax.experimental.pallas import tpu as pltpu
from jax.experimental.pallas import tpu_sc as plsc

B, D, V = 16384, 128, 1000000

NC = 2
NS = 16
NL = 16
NW = NC * NS
BPW = B // NW
SIZES = (32, 96, 192, 192)
STARTS = (0, 32, 128, 320)
SLOT_ROWS = 192
_INV = 1.0 / float(B * D)


def _sc_body(idx_hbm, table_hbm, l_hbm, r_hbm, out_hbm,
             idx_v, alpha_v, lbuf, rbuf, acc_v,
             sem_a0, sem_a1, sem_a2, sem_a3, sem_c0, sem_c1, sem_c2, sem_c3):
    wid = lax.axis_index("s") * NC + lax.axis_index("c")
    base = wid * BPW
    asems = (sem_a0, sem_a1, sem_a2, sem_a3)
    dsems = (sem_c0, sem_c1, sem_c2, sem_c3)

    def fire_alpha(j):
        sl = pl.ds(STARTS[j], SIZES[j])
        return pltpu.async_copy(table_hbm.at[idx_v.at[sl]],
                                alpha_v.at[sl], asems[j])

    d_cps = {}

    def fire(c):
        s = c % 2
        rows = SIZES[c]
        d_cps[c] = (
            pltpu.async_copy(l_hbm.at[pl.ds(base + STARTS[c], rows), :],
                             lbuf.at[s, pl.ds(0, rows), :], dsems[c]),
            pltpu.async_copy(r_hbm.at[pl.ds(base + STARTS[c], rows), :],
                             rbuf.at[s, pl.ds(0, rows), :], dsems[c]),
        )

    fire(0)
    pltpu.sync_copy(idx_hbm.at[pl.ds(wid * BPW, BPW)], idx_v)
    a_cps = {j: fire_alpha(j) for j in range(len(SIZES))}
    fire(1)

    acc = jnp.zeros((NL,), jnp.float32)
    for c in range(len(SIZES)):
        a_cps[c].wait()
        for cp in d_cps[c]:
            cp.wait()
        if c + 2 < len(SIZES):
            fire(c + 2)
        s = c % 2

        def group_body(g, a, s=s, c=c):
            a16 = alpha_v[pl.ds(STARTS[c] + g * NL, NL)]
            row0 = g * NL

            def col_body(gg, a2):
                col = pl.multiple_of(gg * NL, NL)
                for j in range(NL):
                    lv = lbuf[s, row0 + j, pl.ds(col, NL)]
                    rv = rbuf[s, row0 + j, pl.ds(col, NL)]
                    a2 = a2 + jnp.abs(lv - a16[j] * rv)
                return a2

            return lax.fori_loop(0, D // NL, col_body, a, unroll=1)

        acc = lax.fori_loop(0, SIZES[c] // NL, group_body, acc, unroll=1)

    acc_v[...] = acc
    pltpu.sync_copy(acc_v, out_hbm.at[wid])


_sc_loss = pl.kernel(
    _sc_body,
    mesh=plsc.VectorSubcoreMesh(core_axis_name="c", subcore_axis_name="s"),
    out_type=jax.ShapeDtypeStruct((NW, NL), jnp.float32),
    scratch_types=[
        pltpu.VMEM((BPW,), jnp.int32),
        pltpu.VMEM((BPW,), jnp.float32),
        pltpu.VMEM((2, SLOT_ROWS, D), jnp.float32),
        pltpu.VMEM((2, SLOT_ROWS, D), jnp.float32),
        pltpu.VMEM((NL,), jnp.float32),
        pltpu.SemaphoreType.DMA,
        pltpu.SemaphoreType.DMA,
        pltpu.SemaphoreType.DMA,
        pltpu.SemaphoreType.DMA,
        pltpu.SemaphoreType.DMA,
        pltpu.SemaphoreType.DMA,
        pltpu.SemaphoreType.DMA,
        pltpu.SemaphoreType.DMA,
    ],
)


def _fin_body(p_ref, out_ref):
    out_ref[0, 0] = jnp.sum(p_ref[...]) * _INV


_finish = pl.pallas_call(
    _fin_body,
    out_specs=pl.BlockSpec(memory_space=pltpu.SMEM),
    out_shape=jax.ShapeDtypeStruct((1, 1), jnp.float32),
)


def kernel(emd_l, emd_r, player, alpha_weight):
    idx = player.astype(jnp.int32)
    table = alpha_weight.reshape(V)
    parts = _sc_loss(idx, table, emd_l, emd_r)
    return _finish(parts)[0, 0]
